# baseline (speedup 1.0000x reference)
import functools

import jax
import jax.numpy as jnp
import numpy as np
from jax.experimental import pallas as pl
from jax.experimental.pallas import tpu as pltpu

_CONN = [[15, 13], [13, 11], [16, 14], [14, 12], [11, 12], [5, 11], [6, 12],
         [5, 6], [5, 7], [6, 8], [7, 9], [8, 10], [1, 2], [0, 1], [0, 2],
         [1, 3], [2, 4], [3, 5], [4, 6]]
_K = 17
_HID = 256
_INTER = 64
_L = 2
_EPS = 1e-5

_ROW, _COL = [], []
for _s, _d in _CONN:
    _ROW += [_s, _d]
    _COL += [_d, _s]
_E = len(_ROW)
_G = len(_CONN)
_INC = [[e for e, r in enumerate(_ROW) if r == n] for n in range(_K)]
_DEG = [float(len(_INC[n])) for n in range(_K)]
_NBR = [[_COL[e] for e in _INC[n]] for n in range(_K)]


def _silu_u(u):
    return u * jnp.tanh(u) + u


def _sigmoid_u(u):
    return 0.5 * jnp.tanh(u) + 0.5


def _silu(x):
    u = 0.5 * x
    return u * jnp.tanh(u) + u


def _bn_ax0(x3, g, b):
    m = x3.mean(0)
    v = (x3 * x3).mean(0) - m * m
    sc = g * jax.lax.rsqrt(v + _EPS)
    return x3 * sc + (b - m * sc)


def _dotT(a, w):
    return jax.lax.dot_general(a, w, (((1,), (1,)), ((), ())),
                               preferred_element_type=jnp.float32)


def _bdiag2(w):
    o, i = w.shape
    z = jnp.zeros((o, i), w.dtype)
    return jnp.concatenate([jnp.concatenate([w, z], 1),
                            jnp.concatenate([z, w], 1)], 0)


def _row2(v):
    return jnp.concatenate([v, v], 1)


def _gnn_kernel(x_ref, wi_ref, bi_ref, gi_ref, bbi_ref, *lrefs, out_ref):
    n, s, c = x_ref.shape
    h2 = s // 2
    x3 = x_ref[...]
    h = _dotT(x3.reshape(n * s, c), wi_ref[...]) + bi_ref[...]
    x3 = _silu_u(_bn_ax0(h.reshape(n, s, c), 0.5 * gi_ref[...],
                         0.5 * bbi_ref[...]))

    for i in range(_L):
        (ecW, ecg, ecb, bng, bnb, adW, adb, adg, adbb,
         aeW, aeg, aeb, aaW, aab) = lrefs[14 * i:14 * (i + 1)]
        xr = x3
        ew = ecW[...]
        wd = ew[:, :c] - ew[:, c:]
        xf = x3.reshape(n * s, c)
        a3 = _dotT(xf, wd).reshape(n, s, c)
        b3 = _dotT(xf, ew[:, c:]).reshape(n, s, c)
        me = sum((_DEG[nn] / _E) * (a3[nn] + b3[nn]) for nn in range(n))
        sq = sum(_DEG[nn] * (a3[nn] * a3[nn] + b3[nn] * b3[nn])
                 for nn in range(n))
        cross = sum(a3[nn] * sum(b3[j] for j in _NBR[nn]) for nn in range(n))
        ve = (sq + 2.0 * cross) * (1.0 / _E) - me * me
        sce = (0.5 * ecg[...]) * jax.lax.rsqrt(ve + _EPS)
        a3s = a3 * sce
        b3s = b3 * sce + (0.5 * ecb[...] - me * sce)
        h3 = _silu_u(jnp.stack([a3s[r] + b3s[q]
                                for r, q in zip(_ROW, _COL)], 0))
        out3 = jnp.stack([sum(h3[e] for e in _INC[nn]) for nn in range(n)], 0)
        x3 = _bn_ax0(out3, bng[...], bnb[...])

        xdf = _dotT(x3.reshape(n * s, c), adW[...]) + adb[...]
        xdu = xdf.reshape(n, s, _INTER)
        xdfold = jnp.concatenate([xdu[:, :h2, :], xdu[:, h2:, :]], axis=2)
        xd3 = _silu_u(_bn_ax0(xdfold, 0.5 * _row2(adg[...]),
                              0.5 * _row2(adbb[...])))
        xs3 = jnp.stack([(xd3[a] + xd3[b]) * 0.5 for a, b in _CONN], 0)
        xsf = xs3.reshape(_G * h2, 2 * _INTER)
        aw = aeW[...]
        vd = _bdiag2(aw[:, :_INTER] - aw[:, _INTER:])
        v2 = _bdiag2(aw[:, _INTER:])
        a2 = _dotT(xsf, vd).reshape(_G, h2, 2 * _INTER)
        b2 = _dotT(xsf, v2).reshape(_G, h2, 2 * _INTER)
        m2a = a2.mean(0)
        m2b = b2.mean(0)
        v2m = ((a2 * a2).mean(0) - m2a * m2a) + ((b2 * b2).mean(0) - m2b * m2b)
        scale = (0.5 * _row2(aeg[...])) * jax.lax.rsqrt(v2m + _EPS)
        shift = 0.5 * _row2(aeb[...]) - (m2a + m2b) * scale
        as2 = a2 * scale
        bs2 = b2 * scale + shift
        u0 = as2 + bs2[0]
        attg = u0 * jnp.tanh(u0)
        for j in range(1, _G):
            u = as2 + bs2[j]
            attg = attg + u * jnp.tanh(u)
        attg = attg + (float(_G) * as2 + bs2.sum(0))
        aah = _bdiag2(0.5 * aaW[...])
        att = _sigmoid_u(_dotT(attg.reshape(_G * h2, 2 * _INTER), aah)
                         + 0.5 * _row2(aab[...]))
        attmf = att.reshape(_G, h2, 2 * c).mean(0)
        attm = jnp.concatenate([attmf[:, :c], attmf[:, c:]], axis=0)
        x3 = _silu(x3 * attm + xr)
    out_ref[...] = x3


@functools.partial(jax.jit, static_argnames=())
def kernel(keypoint_embeddings, Wi, bi, gi, bbi,
           ecW0, ecg0, ecb0, bng0, bnb0, adW0, adb0, adg0, adbb0,
           aeW0, aeg0, aeb0, aaW0, aab0,
           ecW1, ecg1, ecb1, bng1, bnb1, adW1, adb1, adg1, adbb1,
           aeW1, aeg1, aeb1, aaW1, aab1):
    S = 64
    x = jnp.transpose(keypoint_embeddings, (1, 0, 2))
    b = x.shape[1]

    def row(v):
        return v.reshape(1, -1)

    ops = [x, Wi, row(bi), row(gi), row(bbi)]
    for (ecW, ecg, ecb, bng, bnb, adW, adb, adg, adbb,
         aeW, aeg, aeb, aaW, aab) in (
            (ecW0, ecg0, ecb0, bng0, bnb0, adW0, adb0, adg0, adbb0,
             aeW0, aeg0, aeb0, aaW0, aab0),
            (ecW1, ecg1, ecb1, bng1, bnb1, adW1, adb1, adg1, adbb1,
             aeW1, aeg1, aeb1, aaW1, aab1)):
        ops += [ecW, row(ecg), row(ecb), row(bng), row(bnb),
                adW, row(adb), row(adg), row(adbb),
                aeW, row(aeg), row(aeb), aaW, row(aab)]

    full = lambda arr: pl.BlockSpec(arr.shape, lambda i: (0,) * arr.ndim)
    in_specs = [pl.BlockSpec((_K, S, _HID), lambda i: (0, i, 0))]
    in_specs += [full(o) for o in ops[1:]]

    out = pl.pallas_call(
        lambda *refs: _gnn_kernel(*refs[:-1], out_ref=refs[-1]),
        grid=(b // S,),
        in_specs=in_specs,
        out_specs=pl.BlockSpec((_K, S, _HID), lambda i: (0, i, 0)),
        out_shape=jax.ShapeDtypeStruct((_K, b, _HID), jnp.float32),
        compiler_params=pltpu.CompilerParams(
            dimension_semantics=("parallel",)),
    )(*ops)
    return jnp.transpose(out, (1, 0, 2))


# speedup vs baseline: 1.0611x; 1.0611x over previous
import functools

import jax
import jax.numpy as jnp
import numpy as np
from jax.experimental import pallas as pl
from jax.experimental.pallas import tpu as pltpu

_CONN = [[15, 13], [13, 11], [16, 14], [14, 12], [11, 12], [5, 11], [6, 12],
         [5, 6], [5, 7], [6, 8], [7, 9], [8, 10], [1, 2], [0, 1], [0, 2],
         [1, 3], [2, 4], [3, 5], [4, 6]]
_K = 17
_HID = 256
_INTER = 64
_L = 2
_EPS = 1e-5

_ROW, _COL = [], []
for _s, _d in _CONN:
    _ROW += [_s, _d]
    _COL += [_d, _s]
_E = len(_ROW)
_G = len(_CONN)
_INC = [[e for e, r in enumerate(_ROW) if r == n] for n in range(_K)]
_DEG = [float(len(_INC[n])) for n in range(_K)]
_NBR = [[_COL[e] for e in _INC[n]] for n in range(_K)]


def _silu_u(u):
    return u * jnp.tanh(u) + u


def _sigmoid_u(u):
    return 0.5 * jnp.tanh(u) + 0.5


def _silu(x):
    u = 0.5 * x
    return u * jnp.tanh(u) + u


def _bn_ax0(x3, g, b):
    m = x3.mean(0)
    v = (x3 * x3).mean(0) - m * m
    sc = g * jax.lax.rsqrt(v + _EPS)
    return x3 * sc + (b - m * sc)


def _dotT(a, w):
    return jax.lax.dot_general(a, w, (((1,), (1,)), ((), ())),
                               preferred_element_type=jnp.float32)


def _bdiag2(w):
    o, i = w.shape
    z = jnp.zeros((o, i), w.dtype)
    return jnp.concatenate([jnp.concatenate([w, z], 1),
                            jnp.concatenate([z, w], 1)], 0)


def _row2(v):
    return jnp.concatenate([v, v], 1)


def _gnn_kernel(x_ref, wi_ref, bi_ref, gi_ref, bbi_ref, *lrefs, out_ref):
    n, s, c = x_ref.shape
    h2 = s // 2
    x3 = x_ref[...]
    h = _dotT(x3.reshape(n * s, c), wi_ref[...]) + bi_ref[...]
    x3 = _silu_u(_bn_ax0(h.reshape(n, s, c), 0.5 * gi_ref[...],
                         0.5 * bbi_ref[...]))

    for i in range(_L):
        (ecW, ecg, ecb, bng, bnb, adW, adb, adg, adbb,
         aeW, aeg, aeb, aaW, aab) = lrefs[14 * i:14 * (i + 1)]
        xr = x3
        ew = ecW[...]
        wd = ew[:, :c] - ew[:, c:]
        xf = x3.reshape(n * s, c)
        a3 = _dotT(xf, wd).reshape(n, s, c)
        b3 = _dotT(xf, ew[:, c:]).reshape(n, s, c)
        me = sum((_DEG[nn] / _E) * (a3[nn] + b3[nn]) for nn in range(n))
        sq = sum(_DEG[nn] * (a3[nn] * a3[nn] + b3[nn] * b3[nn])
                 for nn in range(n))
        cross = sum(a3[nn] * sum(b3[j] for j in _NBR[nn]) for nn in range(n))
        ve = (sq + 2.0 * cross) * (1.0 / _E) - me * me
        sce = (0.5 * ecg[...]) * jax.lax.rsqrt(ve + _EPS)
        a3s = a3 * sce
        b3s = b3 * sce + (0.5 * ecb[...] - me * sce)
        h3 = _silu_u(jnp.stack([a3s[r] + b3s[q]
                                for r, q in zip(_ROW, _COL)], 0))
        out3 = jnp.stack([sum(h3[e] for e in _INC[nn]) for nn in range(n)], 0)
        x3 = _bn_ax0(out3, bng[...], bnb[...])

        xdf = _dotT(x3.reshape(n * s, c), adW[...]) + adb[...]
        xdu = xdf.reshape(n, s, _INTER)
        xdfold = jnp.concatenate([xdu[:, :h2, :], xdu[:, h2:, :]], axis=2)
        xd3 = _silu_u(_bn_ax0(xdfold, 0.5 * _row2(adg[...]),
                              0.5 * _row2(adbb[...])))
        xs3 = jnp.stack([(xd3[a] + xd3[b]) * 0.5 for a, b in _CONN], 0)
        xsf = xs3.reshape(_G * h2, 2 * _INTER)
        aw = aeW[...]
        vd = _bdiag2(aw[:, :_INTER] - aw[:, _INTER:])
        v2 = _bdiag2(aw[:, _INTER:])
        a2 = _dotT(xsf, vd).reshape(_G, h2, 2 * _INTER)
        b2 = _dotT(xsf, v2).reshape(_G, h2, 2 * _INTER)
        m2a = a2.mean(0)
        m2b = b2.mean(0)
        v2m = ((a2 * a2).mean(0) - m2a * m2a) + ((b2 * b2).mean(0) - m2b * m2b)
        scale = (0.5 * _row2(aeg[...])) * jax.lax.rsqrt(v2m + _EPS)
        shift = 0.5 * _row2(aeb[...]) - (m2a + m2b) * scale
        as2 = a2 * scale
        bs2 = b2 * scale + shift
        u0 = as2 + bs2[0]
        attg = u0 * jnp.tanh(u0)
        for j in range(1, _G):
            u = as2 + bs2[j]
            attg = attg + u * jnp.tanh(u)
        attg = attg + (float(_G) * as2 + bs2.sum(0))
        aah = _bdiag2(0.5 * aaW[...])
        att = _sigmoid_u(_dotT(attg.reshape(_G * h2, 2 * _INTER), aah)
                         + 0.5 * _row2(aab[...]))
        attmf = att.reshape(_G, h2, 2 * c).mean(0)
        attm = jnp.concatenate([attmf[:, :c], attmf[:, c:]], axis=0)
        x3 = _silu(x3 * attm + xr)
    out_ref[...] = x3


@functools.partial(jax.jit, static_argnames=())
def kernel(keypoint_embeddings, Wi, bi, gi, bbi,
           ecW0, ecg0, ecb0, bng0, bnb0, adW0, adb0, adg0, adbb0,
           aeW0, aeg0, aeb0, aaW0, aab0,
           ecW1, ecg1, ecb1, bng1, bnb1, adW1, adb1, adg1, adbb1,
           aeW1, aeg1, aeb1, aaW1, aab1):
    S = 128
    x = jnp.transpose(keypoint_embeddings, (1, 0, 2))
    b = x.shape[1]

    def row(v):
        return v.reshape(1, -1)

    ops = [x, Wi, row(bi), row(gi), row(bbi)]
    for (ecW, ecg, ecb, bng, bnb, adW, adb, adg, adbb,
         aeW, aeg, aeb, aaW, aab) in (
            (ecW0, ecg0, ecb0, bng0, bnb0, adW0, adb0, adg0, adbb0,
             aeW0, aeg0, aeb0, aaW0, aab0),
            (ecW1, ecg1, ecb1, bng1, bnb1, adW1, adb1, adg1, adbb1,
             aeW1, aeg1, aeb1, aaW1, aab1)):
        ops += [ecW, row(ecg), row(ecb), row(bng), row(bnb),
                adW, row(adb), row(adg), row(adbb),
                aeW, row(aeg), row(aeb), aaW, row(aab)]

    full = lambda arr: pl.BlockSpec(arr.shape, lambda i: (0,) * arr.ndim)
    in_specs = [pl.BlockSpec((_K, S, _HID), lambda i: (0, i, 0))]
    in_specs += [full(o) for o in ops[1:]]

    out = pl.pallas_call(
        lambda *refs: _gnn_kernel(*refs[:-1], out_ref=refs[-1]),
        grid=(b // S,),
        in_specs=in_specs,
        out_specs=pl.BlockSpec((_K, S, _HID), lambda i: (0, i, 0)),
        out_shape=jax.ShapeDtypeStruct((_K, b, _HID), jnp.float32),
        compiler_params=pltpu.CompilerParams(
            dimension_semantics=("parallel",)),
    )(*ops)
    return jnp.transpose(out, (1, 0, 2))


# speedup vs baseline: 1.0613x; 1.0003x over previous
import functools

import jax
import jax.numpy as jnp
from jax.experimental import pallas as pl
from jax.experimental.pallas import tpu as pltpu

_CONN = [[15, 13], [13, 11], [16, 14], [14, 12], [11, 12], [5, 11], [6, 12],
         [5, 6], [5, 7], [6, 8], [7, 9], [8, 10], [1, 2], [0, 1], [0, 2],
         [1, 3], [2, 4], [3, 5], [4, 6]]
_K = 17
_HID = 256
_INTER = 64
_L = 2
_EPS = 1e-5

_ROW, _COL = [], []
for _s, _d in _CONN:
    _ROW += [_s, _d]
    _COL += [_d, _s]
_E = len(_ROW)
_G = len(_CONN)
_INC = [[e for e, r in enumerate(_ROW) if r == n] for n in range(_K)]
_DEG = [float(len(_INC[n])) for n in range(_K)]
_NBR = [[_COL[e] for e in _INC[n]] for n in range(_K)]


def _silu_u(u):
    return u * jnp.tanh(u) + u


def _sigmoid_u(u):
    return 0.5 * jnp.tanh(u) + 0.5


def _silu(x):
    u = 0.5 * x
    return u * jnp.tanh(u) + u


def _bn_ax0(x3, g, b):
    m = x3.mean(0)
    v = (x3 * x3).mean(0) - m * m
    sc = g * jax.lax.rsqrt(v + _EPS)
    return x3 * sc + (b - m * sc)


def _dotT(a, w):
    return jax.lax.dot_general(a, w, (((1,), (1,)), ((), ())),
                               preferred_element_type=jnp.float32)


def _bdiag2(w):
    o, i = w.shape
    z = jnp.zeros((o, i), w.dtype)
    return jnp.concatenate([jnp.concatenate([w, z], 1),
                            jnp.concatenate([z, w], 1)], 0)


def _row2(v):
    return jnp.concatenate([v, v], 1)


def _gnn_kernel(x_ref, wi_ref, bi_ref, gi_ref, bbi_ref, *lrefs, out_ref):
    n, s, c = x_ref.shape
    h2 = s // 2
    x3 = x_ref[...]
    h = _dotT(x3.reshape(n * s, c), wi_ref[...]) + bi_ref[...]
    x3 = _silu_u(_bn_ax0(h.reshape(n, s, c), 0.5 * gi_ref[...],
                         0.5 * bbi_ref[...]))

    for i in range(_L):
        (ecW, ecg, ecb, bng, bnb, adW, adb, adg, adbb,
         aeW, aeg, aeb, aaW, aab) = lrefs[14 * i:14 * (i + 1)]
        xr = x3
        ew = ecW[...]
        wd = ew[:, :c] - ew[:, c:]
        xf = x3.reshape(n * s, c)
        a3 = _dotT(xf, wd).reshape(n, s, c)
        b3 = _dotT(xf, ew[:, c:]).reshape(n, s, c)
        me = sum((_DEG[nn] / _E) * (a3[nn] + b3[nn]) for nn in range(n))
        sq = sum(_DEG[nn] * (a3[nn] * a3[nn] + b3[nn] * b3[nn])
                 for nn in range(n))
        cross = sum(a3[nn] * sum(b3[j] for j in _NBR[nn]) for nn in range(n))
        ve = (sq + 2.0 * cross) * (1.0 / _E) - me * me
        sce = (0.5 * ecg[...]) * jax.lax.rsqrt(ve + _EPS)
        a3s = a3 * sce
        b3s = b3 * sce + (0.5 * ecb[...] - me * sce)
        h3 = _silu_u(jnp.stack([a3s[r] + b3s[q]
                                for r, q in zip(_ROW, _COL)], 0))
        out3 = jnp.stack([sum(h3[e] for e in _INC[nn]) for nn in range(n)], 0)
        x3 = _bn_ax0(out3, bng[...], bnb[...])

        xdf = _dotT(x3.reshape(n * s, c), adW[...]) + adb[...]
        xdu = xdf.reshape(n, s, _INTER)
        xdfold = jnp.concatenate([xdu[:, :h2, :], xdu[:, h2:, :]], axis=2)
        xd3 = _silu_u(_bn_ax0(xdfold, 0.5 * _row2(adg[...]),
                              0.5 * _row2(adbb[...])))
        xs3 = jnp.stack([(xd3[a] + xd3[b]) * 0.5 for a, b in _CONN], 0)
        xsf = xs3.reshape(_G * h2, 2 * _INTER)
        aw = aeW[...]
        vd = _bdiag2(aw[:, :_INTER] - aw[:, _INTER:])
        v2 = _bdiag2(aw[:, _INTER:])
        a2 = _dotT(xsf, vd).reshape(_G, h2, 2 * _INTER)
        b2 = _dotT(xsf, v2).reshape(_G, h2, 2 * _INTER)
        m2a = a2.mean(0)
        m2b = b2.mean(0)
        v2m = ((a2 * a2).mean(0) - m2a * m2a) + ((b2 * b2).mean(0) - m2b * m2b)
        scale = (0.5 * _row2(aeg[...])) * jax.lax.rsqrt(v2m + _EPS)
        shift = 0.5 * _row2(aeb[...]) - (m2a + m2b) * scale
        as2 = a2 * scale
        bs2 = b2 * scale + shift
        u0 = as2 + bs2[0]
        attg = u0 * jnp.tanh(u0)
        for j in range(1, _G):
            u = as2 + bs2[j]
            attg = attg + u * jnp.tanh(u)
        attg = attg + (float(_G) * as2 + bs2.sum(0))
        aah = _bdiag2(0.5 * aaW[...])
        att = _sigmoid_u(_dotT(attg.reshape(_G * h2, 2 * _INTER), aah)
                         + 0.5 * _row2(aab[...]))
        attmf = att.reshape(_G, h2, 2 * c).mean(0)
        attm = jnp.concatenate([attmf[:, :c], attmf[:, c:]], axis=0)
        x3 = _silu(x3 * attm + xr)
    out_ref[...] = x3


@functools.partial(jax.jit, static_argnames=())
def kernel(keypoint_embeddings, Wi, bi, gi, bbi,
           ecW0, ecg0, ecb0, bng0, bnb0, adW0, adb0, adg0, adbb0,
           aeW0, aeg0, aeb0, aaW0, aab0,
           ecW1, ecg1, ecb1, bng1, bnb1, adW1, adb1, adg1, adbb1,
           aeW1, aeg1, aeb1, aaW1, aab1):
    S = 128
    x = jnp.transpose(keypoint_embeddings, (1, 0, 2))
    b = x.shape[1]

    def row(v):
        return v.reshape(1, -1)

    ops = [x, Wi, row(bi), row(gi), row(bbi)]
    for (ecW, ecg, ecb, bng, bnb, adW, adb, adg, adbb,
         aeW, aeg, aeb, aaW, aab) in (
            (ecW0, ecg0, ecb0, bng0, bnb0, adW0, adb0, adg0, adbb0,
             aeW0, aeg0, aeb0, aaW0, aab0),
            (ecW1, ecg1, ecb1, bng1, bnb1, adW1, adb1, adg1, adbb1,
             aeW1, aeg1, aeb1, aaW1, aab1)):
        ops += [ecW, row(ecg), row(ecb), row(bng), row(bnb),
                adW, row(adb), row(adg), row(adbb),
                aeW, row(aeg), row(aeb), aaW, row(aab)]

    full = lambda arr: pl.BlockSpec(arr.shape, lambda i: (0,) * arr.ndim)
    in_specs = [pl.BlockSpec((_K, S, _HID), lambda i: (0, i, 0))]
    in_specs += [full(o) for o in ops[1:]]

    out = pl.pallas_call(
        lambda *refs: _gnn_kernel(*refs[:-1], out_ref=refs[-1]),
        grid=(b // S,),
        in_specs=in_specs,
        out_specs=pl.BlockSpec((_K, S, _HID), lambda i: (0, i, 0)),
        out_shape=jax.ShapeDtypeStruct((_K, b, _HID), jnp.float32),
        compiler_params=pltpu.CompilerParams(
            dimension_semantics=("parallel",)),
    )(*ops)
    return jnp.transpose(out, (1, 0, 2))
